# Initial kernel scaffold; baseline (speedup 1.0000x reference)
import jax
import jax.numpy as jnp
from jax.experimental import pallas as pl


def kernel(efeat, nfeat, edge_index, W1, b1, W2, b2, ln_gamma, ln_beta):
    raise NotImplementedError("write your pallas kernel here")


# speedup vs baseline: 3.4294x; 3.4294x over previous
import functools

import jax
import jax.numpy as jnp
from jax import lax
from jax.experimental import pallas as pl
from jax.experimental.pallas import tpu as pltpu
from jax.experimental.pallas import tpu_sc as plsc

N_NODES = 10000
N_EDGES = 320000
D = 128

NC = 2
NS = 16
NW = NC * NS
E_PER_W = N_EDGES // NW
CHUNK = 80
NCHUNK = E_PER_W // CHUNK
N_PAD = 10240
ROWS_PER_S = N_PAD // NS

_sc_mesh = plsc.VectorSubcoreMesh(core_axis_name="c", subcore_axis_name="s")


@functools.partial(
    pl.kernel,
    out_type=jax.ShapeDtypeStruct((NC, N_PAD, D), jnp.float32),
    mesh=_sc_mesh,
    scratch_types=[
        pltpu.VMEM((NCHUNK, CHUNK), jnp.int32),
        pltpu.VMEM((CHUNK, D), jnp.float32),
        pltpu.VMEM_SHARED((N_PAD, D), jnp.float32),
    ],
)
def _segsum_sc(efeat_hbm, dst_hbm, zeros_hbm, out_hbm, idx_v, rows_v, agg_s):
    c = lax.axis_index("c")
    s = lax.axis_index("s")
    w = c * NS + s

    pltpu.sync_copy(zeros_hbm, agg_s.at[pl.ds(s * ROWS_PER_S, ROWS_PER_S)])
    pltpu.sync_copy(dst_hbm.at[w], idx_v)
    plsc.subcore_barrier()

    def body(j, carry):
        base = w * E_PER_W + j * CHUNK
        pltpu.sync_copy(efeat_hbm.at[pl.ds(base, CHUNK)], rows_v)
        pltpu.sync_copy(rows_v, agg_s.at[idx_v.at[j]], add=True)
        return carry

    lax.fori_loop(0, NCHUNK, body, 0)
    plsc.subcore_barrier()

    pltpu.sync_copy(
        agg_s.at[pl.ds(s * ROWS_PER_S, ROWS_PER_S)],
        out_hbm.at[c, pl.ds(s * ROWS_PER_S, ROWS_PER_S)],
    )


_ROW_BLK = 1000


def _mlp_body(parts_ref, nfeat_ref, w1a_ref, w1b_ref, b1_ref, w2_ref, b2_ref,
              gamma_ref, beta_ref, out_ref):
    agg = parts_ref[0] + parts_ref[1]
    n = nfeat_ref[...]
    h = jnp.dot(agg, w1a_ref[...], preferred_element_type=jnp.float32)
    h = h + jnp.dot(n, w1b_ref[...], preferred_element_type=jnp.float32)
    h = h + b1_ref[...]
    h = h * jax.nn.sigmoid(h)
    h2 = jnp.dot(h, w2_ref[...], preferred_element_type=jnp.float32) + b2_ref[...]
    mean = jnp.mean(h2, axis=-1, keepdims=True)
    var = jnp.mean((h2 - mean) ** 2, axis=-1, keepdims=True)
    y = (h2 - mean) * lax.rsqrt(var + 1e-5) * gamma_ref[...] + beta_ref[...]
    out_ref[...] = y + n


def _mlp_tc(parts, nfeat, w1a, w1b, b1, w2, b2, gamma, beta):
    grid = (N_NODES // _ROW_BLK,)
    full = lambda shape: pl.BlockSpec(shape, lambda i: (0,) * len(shape))
    return pl.pallas_call(
        _mlp_body,
        grid=grid,
        in_specs=[
            pl.BlockSpec((NC, _ROW_BLK, D), lambda i: (0, i, 0)),
            pl.BlockSpec((_ROW_BLK, D), lambda i: (i, 0)),
            full((D, D)), full((D, D)), full((1, D)),
            full((D, D)), full((1, D)), full((1, D)), full((1, D)),
        ],
        out_specs=pl.BlockSpec((_ROW_BLK, D), lambda i: (i, 0)),
        out_shape=jax.ShapeDtypeStruct((N_NODES, D), jnp.float32),
    )(parts, nfeat, w1a, w1b, b1, w2, b2, gamma, beta)


def kernel(efeat, nfeat, edge_index, W1, b1, W2, b2, ln_gamma, ln_beta):
    dst = edge_index[1].astype(jnp.int32).reshape(NW, NCHUNK, CHUNK)
    zeros = jnp.zeros((ROWS_PER_S, D), jnp.float32)
    parts = _segsum_sc(efeat, dst, zeros)
    nfeat_new = _mlp_tc(
        parts, nfeat,
        W1[:D], W1[D:], b1.reshape(1, D),
        W2, b2.reshape(1, D),
        ln_gamma.reshape(1, D), ln_beta.reshape(1, D),
    )
    return (efeat, nfeat_new)


# speedup vs baseline: 4.5440x; 1.3250x over previous
import functools

import jax
import jax.numpy as jnp
from jax import lax
from jax.experimental import pallas as pl
from jax.experimental.pallas import tpu as pltpu
from jax.experimental.pallas import tpu_sc as plsc

N_NODES = 10000
N_EDGES = 320000
D = 128

NC = 2
NS = 16
NW = NC * NS
E_PER_W = N_EDGES // NW
CHUNK = 80
NCHUNK = E_PER_W // CHUNK
N_PAD = 10240
ROWS_PER_S = N_PAD // NS

_sc_mesh = plsc.VectorSubcoreMesh(core_axis_name="c", subcore_axis_name="s")


@functools.partial(
    pl.kernel,
    out_type=jax.ShapeDtypeStruct((NC, N_PAD, D), jnp.float32),
    mesh=_sc_mesh,
    scratch_types=[
        pltpu.VMEM((NCHUNK, CHUNK), jnp.int32),
        pltpu.VMEM((CHUNK, D), jnp.float32),
        pltpu.VMEM((CHUNK, D), jnp.float32),
        pltpu.VMEM_SHARED((N_PAD, D), jnp.float32),
        pltpu.SemaphoreType.DMA,
        pltpu.SemaphoreType.DMA,
    ],
)
def _segsum_sc(efeat_hbm, dst_hbm, zeros_hbm, out_hbm, idx_v, rows0, rows1,
               agg_s, sem0, sem1):
    c = lax.axis_index("c")
    s = lax.axis_index("s")
    w = c * NS + s
    e_base = w * E_PER_W

    pltpu.sync_copy(zeros_hbm, agg_s.at[pl.ds(s * ROWS_PER_S, ROWS_PER_S)])
    pltpu.sync_copy(dst_hbm.at[w], idx_v)
    plsc.subcore_barrier()

    def load(j, buf, sem):
        pltpu.async_copy(efeat_hbm.at[pl.ds(e_base + j * CHUNK, CHUNK)], buf, sem)

    def wait(buf, sem):
        pltpu.make_async_copy(efeat_hbm.at[pl.ds(0, CHUNK)], buf, sem).wait()

    def scat(j, buf):
        pltpu.sync_copy(buf, agg_s.at[idx_v.at[j]], add=True)

    load(0, rows0, sem0)
    load(1, rows1, sem1)

    def body(jj, carry):
        j = 2 * jj
        wait(rows0, sem0)
        scat(j, rows0)
        load(j + 2, rows0, sem0)
        wait(rows1, sem1)
        scat(j + 1, rows1)
        load(j + 3, rows1, sem1)
        return carry

    lax.fori_loop(0, (NCHUNK - 3) // 2, body, 0)
    wait(rows0, sem0)
    scat(NCHUNK - 3, rows0)
    load(NCHUNK - 1, rows0, sem0)
    wait(rows1, sem1)
    scat(NCHUNK - 2, rows1)
    wait(rows0, sem0)
    scat(NCHUNK - 1, rows0)
    plsc.subcore_barrier()

    pltpu.sync_copy(
        agg_s.at[pl.ds(s * ROWS_PER_S, ROWS_PER_S)],
        out_hbm.at[c, pl.ds(s * ROWS_PER_S, ROWS_PER_S)],
    )


_ROW_BLK = 1000


def _mlp_body(parts_ref, nfeat_ref, w1a_ref, w1b_ref, b1_ref, w2_ref, b2_ref,
              gamma_ref, beta_ref, out_ref):
    agg = parts_ref[0] + parts_ref[1]
    n = nfeat_ref[...]
    h = jnp.dot(agg, w1a_ref[...], preferred_element_type=jnp.float32)
    h = h + jnp.dot(n, w1b_ref[...], preferred_element_type=jnp.float32)
    h = h + b1_ref[...]
    h = h * jax.nn.sigmoid(h)
    h2 = jnp.dot(h, w2_ref[...], preferred_element_type=jnp.float32) + b2_ref[...]
    mean = jnp.mean(h2, axis=-1, keepdims=True)
    var = jnp.mean((h2 - mean) ** 2, axis=-1, keepdims=True)
    y = (h2 - mean) * lax.rsqrt(var + 1e-5) * gamma_ref[...] + beta_ref[...]
    out_ref[...] = y + n


def _mlp_tc(parts, nfeat, w1a, w1b, b1, w2, b2, gamma, beta):
    grid = (N_NODES // _ROW_BLK,)
    full = lambda shape: pl.BlockSpec(shape, lambda i: (0,) * len(shape))
    return pl.pallas_call(
        _mlp_body,
        grid=grid,
        in_specs=[
            pl.BlockSpec((NC, _ROW_BLK, D), lambda i: (0, i, 0)),
            pl.BlockSpec((_ROW_BLK, D), lambda i: (i, 0)),
            full((D, D)), full((D, D)), full((1, D)),
            full((D, D)), full((1, D)), full((1, D)), full((1, D)),
        ],
        out_specs=pl.BlockSpec((_ROW_BLK, D), lambda i: (i, 0)),
        out_shape=jax.ShapeDtypeStruct((N_NODES, D), jnp.float32),
    )(parts, nfeat, w1a, w1b, b1, w2, b2, gamma, beta)


def kernel(efeat, nfeat, edge_index, W1, b1, W2, b2, ln_gamma, ln_beta):
    dst = edge_index[1].astype(jnp.int32).reshape(NW, NCHUNK, CHUNK)
    zeros = jnp.zeros((ROWS_PER_S, D), jnp.float32)
    parts = _segsum_sc(efeat, dst, zeros)
    nfeat_new = _mlp_tc(
        parts, nfeat,
        W1[:D], W1[D:], b1.reshape(1, D),
        W2, b2.reshape(1, D),
        ln_gamma.reshape(1, D), ln_beta.reshape(1, D),
    )
    return (efeat, nfeat_new)
